# baseline (speedup 1.0000x reference)
import functools

import jax
import jax.numpy as jnp
from jax import lax
from jax.experimental import pallas as pl
from jax.experimental.pallas import tpu as pltpu
from jax.experimental.pallas import tpu_sc as plsc

N = 10000
E = 320000
B = 64
D_IN = 128
HID = 256
NL = 4

NC = 2
NS = 16
CHUNK = 112
G = 192
G0 = 96
E_PAD = NS * G * CHUNK
NB = 3
IB = 6
IBG = 2 * IB
ACC_ROWS = 10240


@functools.lru_cache(maxsize=None)
def _make_spmm(dh, g):
    mesh = plsc.VectorSubcoreMesh(core_axis_name="c", subcore_axis_name="s",
                                  num_cores=NC, num_subcores=NS)

    ib = IB
    ibg = IBG
    ngroups = g // ibg

    @functools.partial(
        pl.kernel,
        out_type=jax.ShapeDtypeStruct((NC, ACC_ROWS, dh), jnp.float32),
        mesh=mesh,
        scratch_types=(
            [pltpu.VMEM((ib, 2, CHUNK), jnp.int32) for _ in range(2)]
            + [pltpu.VMEM((CHUNK, dh), jnp.float32) for _ in range(NB)]
            + [pltpu.VMEM_SHARED((ACC_ROWS, dh), jnp.float32)]
            + [pltpu.SemaphoreType.DMA for _ in range(2 * NB)]
        ),
    )
    def spmm(h2, idx, out, *refs):
        idxbs = refs[0:2]
        rows = refs[2:2 + NB]
        acc = refs[2 + NB]
        gsem = refs[3 + NB:3 + 2 * NB]
        ssem = refs[3 + 2 * NB:3 + 3 * NB]
        idxb0, idxb1 = idxbs
        rows0 = rows[0]
        c = lax.axis_index("c")
        sid = lax.axis_index("s")

        def zrow(r, carry):
            for j in range(dh // 16):
                rows0[r, pl.ds(j * 16, 16)] = jnp.zeros((16,), jnp.float32)
            return carry

        lax.fori_loop(0, CHUNK, zrow, 0)
        zrows = ACC_ROWS // NS
        zbase = sid * zrows
        for z in range(zrows // CHUNK):
            pltpu.sync_copy(rows0, acc.at[pl.ds(zbase + z * CHUNK, CHUNK)])
        zrem = zrows % CHUNK
        if zrem:
            pltpu.sync_copy(rows0.at[pl.ds(0, zrem)],
                            acc.at[pl.ds(zbase + zrows - zrem, zrem)])
        plsc.subcore_barrier()

        def wait_rows(b):
            pltpu.make_async_copy(h2.at[pl.ds(0, CHUNK)], rows[b], gsem[b]).wait()

        def wait_scat(b):
            pltpu.make_async_copy(rows[b], acc.at[pl.ds(0, CHUNK)], ssem[b]).wait()

        pltpu.sync_copy(idx.at[c, sid, pl.ds(0, ib)], idxb0)
        pltpu.sync_copy(idx.at[c, sid, pl.ds(ib, ib)], idxb1)
        pltpu.async_copy(h2.at[idxb0.at[0, 0]], rows[0], gsem[0])
        pltpu.async_copy(h2.at[idxb0.at[1, 0]], rows[1], gsem[1])

        def group(t, carry):
            for j2 in range(2):
                blk = idxbs[j2]
                for u in range(ib):
                    p = j2 * ib + u
                    b = p % NB
                    wait_rows(b)
                    pltpu.async_copy(rows[b], acc.at[blk.at[u, 1]], ssem[b],
                                     add=True)
                    if p >= 1:
                        wait_scat((p - 1) % NB)
                    if p < ibg - 2:
                        p2 = p + 2
                        b2 = p2 % NB
                        blk2 = idxbs[p2 // ib]
                        pltpu.async_copy(h2.at[blk2.at[p2 % ib, 0]], rows[b2],
                                         gsem[b2])

            @pl.when(t < ngroups - 1)
            def _prefetch():
                wait_scat((ibg - 1) % NB)
                base = (t + 1) * ibg
                pltpu.sync_copy(idx.at[c, sid, pl.ds(base, ib)], idxb0)
                pltpu.sync_copy(idx.at[c, sid, pl.ds(base + ib, ib)], idxb1)
                pltpu.async_copy(h2.at[idxb0.at[0, 0]], rows[0], gsem[0])
                pltpu.async_copy(h2.at[idxb0.at[1, 0]], rows[1], gsem[1])

            return carry

        lax.fori_loop(0, ngroups, group, 0)
        wait_scat((ibg - 1) % NB)
        plsc.subcore_barrier()

        wrows = ACC_ROWS // NS
        pltpu.sync_copy(acc.at[pl.ds(sid * wrows, wrows)],
                        out.at[c, pl.ds(sid * wrows, wrows)])

    return spmm


def _spmm_call(h2, idx, dh):
    return _make_spmm(dh, idx.shape[2])(h2, idx)


def _mlp_body(p0, p1, h, w1, b1, g1, be1, w2, b2, go, beo, eps, out, *, combine):
    a = p0[pl.ds(0, N)]
    b = p1[pl.ds(0, N)]
    pooled = a + b if combine == 'sum' else jnp.concatenate([a, b], axis=-1)
    z = pooled + (1.0 + eps[0, 0]) * h[...]
    t = jnp.dot(z, w1[...], preferred_element_type=jnp.float32) + b1[...]
    m = jnp.mean(t, axis=0, keepdims=True)
    v = jnp.mean((t - m) * (t - m), axis=0, keepdims=True)
    t = (t - m) * lax.rsqrt(v + 1e-5) * g1[...] + be1[...]
    t = jnp.maximum(t, 0.0)
    u = jnp.dot(t, w2[...], preferred_element_type=jnp.float32) + b2[...]
    m2 = jnp.mean(u, axis=0, keepdims=True)
    v2 = jnp.mean((u - m2) * (u - m2), axis=0, keepdims=True)
    u = (u - m2) * lax.rsqrt(v2 + 1e-5) * go[...] + beo[...]
    out[...] = jnp.maximum(u, 0.0)


def _mlp_call(combine, p0, p1, h, w1, b1, g1, be1, w2, b2, go, beo, eps):
    return pl.pallas_call(
        functools.partial(_mlp_body, combine=combine),
        out_shape=jax.ShapeDtypeStruct((N, HID), jnp.float32),
    )(p0, p1, h, w1, b1, g1, be1, w2, b2, go, beo, eps)


def _gsum_body(gid, h, out):
    ids = gid[...]
    iot = lax.broadcasted_iota(jnp.int32, (B, N), 0)
    onehot = (ids == iot).astype(jnp.float32)
    out[...] = jnp.dot(onehot, h[...], preferred_element_type=jnp.float32)


def _gsum_call(gid, h):
    return pl.pallas_call(
        _gsum_body,
        out_shape=jax.ShapeDtypeStruct((B, h.shape[1]), jnp.float32),
    )(gid, h)


def _head_body(gid, sx, s1, s2, s3, s4, dke, pw, pb, out):
    ids = gid[...]
    iot = lax.broadcasted_iota(jnp.int32, (B, N), 0)
    onehot = (ids == iot).astype(jnp.float32)
    counts = jnp.sum(onehot, axis=1, keepdims=True)
    sums = jnp.concatenate(
        [sx[...], s1[...], s2[...], s3[...], s4[...]], axis=-1)
    means = sums / jnp.maximum(counts, 1.0)
    rep = jnp.dot(means, dke[...], preferred_element_type=jnp.float32)
    out[...] = jnp.dot(rep, pw[...], preferred_element_type=jnp.float32) + pb[...]


def _head_call(gid, sx, s1, s2, s3, s4, dke, pw, pb):
    return pl.pallas_call(
        _head_body,
        out_shape=jax.ShapeDtypeStruct((B, 10), jnp.float32),
    )(gid, sx, s1, s2, s3, s4, dke, pw, pb)


def kernel(x, params, edge_index, graph_ids):
    src = edge_index[1].astype(jnp.int32)
    dst = edge_index[0].astype(jnp.int32)

    npad = E_PAD - E
    pad_i = jnp.arange(npad, dtype=jnp.int32)
    src_p = jnp.concatenate([src, pad_i % 1024])
    dst_p = jnp.concatenate([dst, N + (pad_i % (ACC_ROWS - N))])
    idx0 = jnp.concatenate([src_p.reshape(NC, NS, G0, 1, CHUNK),
                            dst_p.reshape(NC, NS, G0, 1, CHUNK)], axis=3)
    dst_q = dst_p.reshape(NS, G, 1, CHUNK)
    idx1 = jnp.stack([
        jnp.concatenate([(2 * src_p + c).reshape(NS, G, 1, CHUNK), dst_q], axis=2)
        for c in range(NC)])

    gid = graph_ids.reshape(1, N).astype(jnp.int32)
    h = x
    sums = [_gsum_call(gid, x)]
    for l in range(NL):
        if l == 0:
            pooled = _spmm_call(h, idx0, D_IN)
            combine = 'sum'
        else:
            pooled = _spmm_call(h.reshape(2 * N, HID // 2), idx1, HID // 2)
            combine = 'concat'
        h = _mlp_call(
            combine, pooled[0], pooled[1], h,
            params[f'l{l}_W1'], params[f'l{l}_b1'].reshape(1, HID),
            params[f'l{l}_g1'].reshape(1, HID), params[f'l{l}_be1'].reshape(1, HID),
            params[f'l{l}_W2'], params[f'l{l}_b2'].reshape(1, HID),
            params[f'l{l}_go'].reshape(1, HID), params[f'l{l}_beo'].reshape(1, HID),
            params['eps'][l].reshape(1, 1),
        )
        sums.append(_gsum_call(gid, h))

    return _head_call(
        gid, sums[0], sums[1], sums[2], sums[3], sums[4],
        params['dke_W'], params['pred_W'], params['pred_b'].reshape(1, 10),
    )


# speedup vs baseline: 1.0705x; 1.0705x over previous
import functools

import jax
import jax.numpy as jnp
from jax import lax
from jax.experimental import pallas as pl
from jax.experimental.pallas import tpu as pltpu
from jax.experimental.pallas import tpu_sc as plsc

N = 10000
E = 320000
B = 64
D_IN = 128
HID = 256
NL = 4

NC = 2
NS = 16
CHUNK = 128
EPW = 20480
E_PAD = NS * EPW
G = EPW // CHUNK
NB = 2
ACC_ROWS = 10240


@functools.lru_cache(maxsize=None)
def _make_spmm(dh, g):
    mesh = plsc.VectorSubcoreMesh(core_axis_name="c", subcore_axis_name="s",
                                  num_cores=NC, num_subcores=NS)

    ib = 8
    ibg = 2 * ib
    ngroups = g // ibg

    @functools.partial(
        pl.kernel,
        out_type=jax.ShapeDtypeStruct((NC, ACC_ROWS, dh), jnp.float32),
        mesh=mesh,
        scratch_types=(
            [pltpu.VMEM((ib, 2, CHUNK), jnp.int32) for _ in range(2)]
            + [pltpu.VMEM((CHUNK, dh), jnp.float32) for _ in range(NB)]
            + [pltpu.VMEM_SHARED((ACC_ROWS, dh), jnp.float32)]
            + [pltpu.SemaphoreType.DMA for _ in range(2 * NB)]
        ),
    )
    def spmm(h2, idx, out, *refs):
        idxbs = refs[0:2]
        rows = refs[2:2 + NB]
        acc = refs[2 + NB]
        gsem = refs[3 + NB:3 + 2 * NB]
        ssem = refs[3 + 2 * NB:3 + 3 * NB]
        idxb0, idxb1 = idxbs
        rows0 = rows[0]
        c = lax.axis_index("c")
        sid = lax.axis_index("s")

        def zrow(r, carry):
            for j in range(dh // 16):
                rows0[r, pl.ds(j * 16, 16)] = jnp.zeros((16,), jnp.float32)
            return carry

        lax.fori_loop(0, CHUNK, zrow, 0)
        zrows = ACC_ROWS // NS
        for z in range(zrows // CHUNK):
            pltpu.sync_copy(rows0, acc.at[pl.ds(sid * zrows + z * CHUNK, CHUNK)])
        plsc.subcore_barrier()

        def wait_rows(b):
            pltpu.make_async_copy(h2.at[pl.ds(0, CHUNK)], rows[b], gsem[b]).wait()

        def wait_scat(b):
            pltpu.make_async_copy(rows[b], acc.at[pl.ds(0, CHUNK)], ssem[b]).wait()

        pltpu.sync_copy(idx.at[c, sid, pl.ds(0, ib)], idxb0)
        pltpu.sync_copy(idx.at[c, sid, pl.ds(ib, ib)], idxb1)
        pltpu.async_copy(h2.at[idxb0.at[0, 0]], rows[0], gsem[0])
        pltpu.async_copy(h2.at[idxb0.at[1, 0]], rows[1], gsem[1])

        def group(t, carry):
            for j2 in range(2):
                blk = idxbs[j2]
                for u in range(ib):
                    p = j2 * ib + u
                    b = p % NB
                    wait_rows(b)
                    pltpu.sync_copy(rows[b], acc.at[blk.at[u, 1]], add=True)
                    if p < ibg - 2:
                        p2 = p + 2
                        blk2 = idxbs[p2 // ib]
                        pltpu.async_copy(h2.at[blk2.at[p2 % ib, 0]], rows[b],
                                         gsem[b])

            @pl.when(t < ngroups - 1)
            def _prefetch():
                base = (t + 1) * ibg
                pltpu.sync_copy(idx.at[c, sid, pl.ds(base, ib)], idxb0)
                pltpu.sync_copy(idx.at[c, sid, pl.ds(base + ib, ib)], idxb1)
                pltpu.async_copy(h2.at[idxb0.at[0, 0]], rows[0], gsem[0])
                pltpu.async_copy(h2.at[idxb0.at[1, 0]], rows[1], gsem[1])

            return carry

        lax.fori_loop(0, ngroups, group, 0)
        plsc.subcore_barrier()

        wrows = ACC_ROWS // NS
        pltpu.sync_copy(acc.at[pl.ds(sid * wrows, wrows)],
                        out.at[c, pl.ds(sid * wrows, wrows)])

    return spmm


def _spmm_call(h2, idx, dh):
    return _make_spmm(dh, idx.shape[2])(h2, idx)


def _mlp_body(gid, p0, p1, h, w1, b1, g1, be1, w2, b2, go, beo, eps, out,
              gout, *, combine):
    a = p0[pl.ds(0, N)]
    b = p1[pl.ds(0, N)]
    pooled = a + b if combine == 'sum' else jnp.concatenate([a, b], axis=-1)
    z = pooled + (1.0 + eps[0, 0]) * h[...]
    t = jnp.dot(z, w1[...], preferred_element_type=jnp.float32) + b1[...]
    m = jnp.mean(t, axis=0, keepdims=True)
    v = jnp.mean((t - m) * (t - m), axis=0, keepdims=True)
    t = (t - m) * lax.rsqrt(v + 1e-5) * g1[...] + be1[...]
    t = jnp.maximum(t, 0.0)
    u = jnp.dot(t, w2[...], preferred_element_type=jnp.float32) + b2[...]
    m2 = jnp.mean(u, axis=0, keepdims=True)
    v2 = jnp.mean((u - m2) * (u - m2), axis=0, keepdims=True)
    u = (u - m2) * lax.rsqrt(v2 + 1e-5) * go[...] + beo[...]
    hn = jnp.maximum(u, 0.0)
    out[...] = hn
    iot = lax.broadcasted_iota(jnp.int32, (B, N), 0)
    onehot = (gid[...] == iot).astype(jnp.float32)
    gout[...] = jnp.dot(onehot, hn, preferred_element_type=jnp.float32)


def _mlp_call(combine, gid, p0, p1, h, w1, b1, g1, be1, w2, b2, go, beo, eps):
    return pl.pallas_call(
        functools.partial(_mlp_body, combine=combine),
        out_shape=(jax.ShapeDtypeStruct((N, HID), jnp.float32),
                   jax.ShapeDtypeStruct((B, HID), jnp.float32)),
    )(gid, p0, p1, h, w1, b1, g1, be1, w2, b2, go, beo, eps)


def _gsum_body(gid, h, out):
    ids = gid[...]
    iot = lax.broadcasted_iota(jnp.int32, (B, N), 0)
    onehot = (ids == iot).astype(jnp.float32)
    out[...] = jnp.dot(onehot, h[...], preferred_element_type=jnp.float32)


def _gsum_call(gid, h):
    return pl.pallas_call(
        _gsum_body,
        out_shape=jax.ShapeDtypeStruct((B, h.shape[1]), jnp.float32),
    )(gid, h)


def _head_body(gid, sx, s1, s2, s3, s4, dke, pw, pb, out):
    ids = gid[...]
    iot = lax.broadcasted_iota(jnp.int32, (B, N), 0)
    onehot = (ids == iot).astype(jnp.float32)
    counts = jnp.sum(onehot, axis=1, keepdims=True)
    sums = jnp.concatenate(
        [sx[...], s1[...], s2[...], s3[...], s4[...]], axis=-1)
    means = sums / jnp.maximum(counts, 1.0)
    rep = jnp.dot(means, dke[...], preferred_element_type=jnp.float32)
    out[...] = jnp.dot(rep, pw[...], preferred_element_type=jnp.float32) + pb[...]


def _head_call(gid, sx, s1, s2, s3, s4, dke, pw, pb):
    return pl.pallas_call(
        _head_body,
        out_shape=jax.ShapeDtypeStruct((B, 10), jnp.float32),
    )(gid, sx, s1, s2, s3, s4, dke, pw, pb)


def kernel(x, params, edge_index, graph_ids):
    src = edge_index[1].astype(jnp.int32)
    dst = edge_index[0].astype(jnp.int32)

    npad = E_PAD - E
    pad_i = jnp.arange(npad, dtype=jnp.int32)
    src_p = jnp.concatenate([src, pad_i % 1024])
    dst_p = jnp.concatenate([dst, N + (pad_i % (ACC_ROWS - N))])
    g0 = E_PAD // (NC * NS * CHUNK)
    idx0 = jnp.concatenate([src_p.reshape(NC, NS, g0, 1, CHUNK),
                            dst_p.reshape(NC, NS, g0, 1, CHUNK)], axis=3)
    dst_q = dst_p.reshape(NS, G, 1, CHUNK)
    idx1 = jnp.stack([
        jnp.concatenate([(2 * src_p + c).reshape(NS, G, 1, CHUNK), dst_q], axis=2)
        for c in range(NC)])

    gid = graph_ids.reshape(1, N).astype(jnp.int32)
    h = x
    sums = [_gsum_call(gid, x)]
    for l in range(NL):
        if l == 0:
            pooled = _spmm_call(h, idx0, D_IN)
            combine = 'sum'
        else:
            pooled = _spmm_call(h.reshape(2 * N, HID // 2), idx1, HID // 2)
            combine = 'concat'
        h, s = _mlp_call(
            combine, gid, pooled[0], pooled[1], h,
            params[f'l{l}_W1'], params[f'l{l}_b1'].reshape(1, HID),
            params[f'l{l}_g1'].reshape(1, HID), params[f'l{l}_be1'].reshape(1, HID),
            params[f'l{l}_W2'], params[f'l{l}_b2'].reshape(1, HID),
            params[f'l{l}_go'].reshape(1, HID), params[f'l{l}_beo'].reshape(1, HID),
            params['eps'][l].reshape(1, 1),
        )
        sums.append(s)

    return _head_call(
        gid, sums[0], sums[1], sums[2], sums[3], sums[4],
        params['dke_W'], params['pred_W'], params['pred_b'].reshape(1, 10),
    )


# speedup vs baseline: 1.0888x; 1.0171x over previous
import functools

import jax
import jax.numpy as jnp
from jax import lax
from jax.experimental import pallas as pl
from jax.experimental.pallas import tpu as pltpu
from jax.experimental.pallas import tpu_sc as plsc

N = 10000
E = 320000
B = 64
D_IN = 128
HID = 256
NL = 4

NC = 2
NS = 16
CHUNK = 128
EPW = 20480
E_PAD = NS * EPW
G = EPW // CHUNK
NB = 2
ACC_ROWS = 10240


@functools.lru_cache(maxsize=None)
def _make_spmm(dh, g):
    mesh = plsc.VectorSubcoreMesh(core_axis_name="c", subcore_axis_name="s",
                                  num_cores=NC, num_subcores=NS)

    ib = 8
    ibg = 2 * ib
    ngroups = g // ibg

    @functools.partial(
        pl.kernel,
        out_type=jax.ShapeDtypeStruct((NC, ACC_ROWS, dh), jnp.float32),
        mesh=mesh,
        scratch_types=(
            [pltpu.VMEM((ib, 2, CHUNK), jnp.int32) for _ in range(2)]
            + [pltpu.VMEM((CHUNK, dh), jnp.float32) for _ in range(NB)]
            + [pltpu.VMEM_SHARED((ACC_ROWS, dh), jnp.float32)]
            + [pltpu.SemaphoreType.DMA for _ in range(2 * NB + 1)]
        ),
    )
    def spmm(h2, idx, out, *refs):
        idxbs = refs[0:2]
        rows = refs[2:2 + NB]
        acc = refs[2 + NB]
        gsem = refs[3 + NB:3 + 2 * NB]
        ssem = refs[3 + 2 * NB:3 + 3 * NB]
        isem = refs[2 + 3 * NB]
        idxb0, idxb1 = idxbs
        rows0 = rows[0]
        c = lax.axis_index("c")
        sid = lax.axis_index("s")

        def zrow(r, carry):
            for j in range(dh // 16):
                rows0[r, pl.ds(j * 16, 16)] = jnp.zeros((16,), jnp.float32)
            return carry

        lax.fori_loop(0, CHUNK, zrow, 0)
        zrows = ACC_ROWS // NS
        for z in range(zrows // CHUNK):
            pltpu.sync_copy(rows0, acc.at[pl.ds(sid * zrows + z * CHUNK, CHUNK)])
        plsc.subcore_barrier()

        def wait_rows(b):
            pltpu.make_async_copy(h2.at[pl.ds(0, CHUNK)], rows[b], gsem[b]).wait()

        def wait_scat(b):
            pltpu.make_async_copy(rows[b], acc.at[pl.ds(0, CHUNK)], ssem[b]).wait()

        pltpu.sync_copy(idx.at[c, sid, pl.ds(0, ib)], idxb0)
        pltpu.sync_copy(idx.at[c, sid, pl.ds(ib, ib)], idxb1)
        pltpu.async_copy(h2.at[idxb0.at[0, 0]], rows[0], gsem[0])
        pltpu.async_copy(h2.at[idxb0.at[1, 0]], rows[1], gsem[1])

        def group(t, carry):
            for j2 in range(2):
                blk = idxbs[j2]
                if j2 == 1:
                    @pl.when(t < ngroups - 1)
                    def _pre0():
                        pltpu.async_copy(
                            idx.at[c, sid, pl.ds((t + 1) * ibg, ib)],
                            idxb0, isem)
                for u in range(ib):
                    p = j2 * ib + u
                    b = p % NB
                    wait_rows(b)
                    pltpu.sync_copy(rows[b], acc.at[blk.at[u, 1]], add=True)
                    if p < ibg - 2:
                        p2 = p + 2
                        blk2 = idxbs[p2 // ib]
                        pltpu.async_copy(h2.at[blk2.at[p2 % ib, 0]], rows[b],
                                         gsem[b])

            @pl.when(t < ngroups - 1)
            def _prefetch():
                base = (t + 1) * ibg
                pltpu.make_async_copy(idx.at[c, sid, pl.ds(base, ib)],
                                      idxb0, isem).wait()
                pltpu.sync_copy(idx.at[c, sid, pl.ds(base + ib, ib)], idxb1)
                pltpu.async_copy(h2.at[idxb0.at[0, 0]], rows[0], gsem[0])
                pltpu.async_copy(h2.at[idxb0.at[1, 0]], rows[1], gsem[1])

            return carry

        lax.fori_loop(0, ngroups, group, 0)
        plsc.subcore_barrier()

        wrows = ACC_ROWS // NS
        pltpu.sync_copy(acc.at[pl.ds(sid * wrows, wrows)],
                        out.at[c, pl.ds(sid * wrows, wrows)])

    return spmm


def _spmm_call(h2, idx, dh):
    return _make_spmm(dh, idx.shape[2])(h2, idx)


def _mlp_body(gid, p0, p1, h, w1, b1, g1, be1, w2, b2, go, beo, eps, out,
              gout, *, combine):
    a = p0[pl.ds(0, N)]
    b = p1[pl.ds(0, N)]
    pooled = a + b if combine == 'sum' else jnp.concatenate([a, b], axis=-1)
    z = pooled + (1.0 + eps[0, 0]) * h[...]
    t = jnp.dot(z, w1[...], preferred_element_type=jnp.float32) + b1[...]
    m = jnp.mean(t, axis=0, keepdims=True)
    v = jnp.mean((t - m) * (t - m), axis=0, keepdims=True)
    t = (t - m) * lax.rsqrt(v + 1e-5) * g1[...] + be1[...]
    t = jnp.maximum(t, 0.0)
    u = jnp.dot(t, w2[...], preferred_element_type=jnp.float32) + b2[...]
    m2 = jnp.mean(u, axis=0, keepdims=True)
    v2 = jnp.mean((u - m2) * (u - m2), axis=0, keepdims=True)
    u = (u - m2) * lax.rsqrt(v2 + 1e-5) * go[...] + beo[...]
    hn = jnp.maximum(u, 0.0)
    out[...] = hn
    iot = lax.broadcasted_iota(jnp.int32, (B, N), 0)
    onehot = (gid[...] == iot).astype(jnp.float32)
    gout[...] = jnp.dot(onehot, hn, preferred_element_type=jnp.float32)


def _mlp_call(combine, gid, p0, p1, h, w1, b1, g1, be1, w2, b2, go, beo, eps):
    return pl.pallas_call(
        functools.partial(_mlp_body, combine=combine),
        out_shape=(jax.ShapeDtypeStruct((N, HID), jnp.float32),
                   jax.ShapeDtypeStruct((B, HID), jnp.float32)),
    )(gid, p0, p1, h, w1, b1, g1, be1, w2, b2, go, beo, eps)


def _gsum_body(gid, h, out):
    ids = gid[...]
    iot = lax.broadcasted_iota(jnp.int32, (B, N), 0)
    onehot = (ids == iot).astype(jnp.float32)
    out[...] = jnp.dot(onehot, h[...], preferred_element_type=jnp.float32)


def _gsum_call(gid, h):
    return pl.pallas_call(
        _gsum_body,
        out_shape=jax.ShapeDtypeStruct((B, h.shape[1]), jnp.float32),
    )(gid, h)


def _head_body(gid, sx, s1, s2, s3, s4, dke, pw, pb, out):
    ids = gid[...]
    iot = lax.broadcasted_iota(jnp.int32, (B, N), 0)
    onehot = (ids == iot).astype(jnp.float32)
    counts = jnp.sum(onehot, axis=1, keepdims=True)
    sums = jnp.concatenate(
        [sx[...], s1[...], s2[...], s3[...], s4[...]], axis=-1)
    means = sums / jnp.maximum(counts, 1.0)
    rep = jnp.dot(means, dke[...], preferred_element_type=jnp.float32)
    out[...] = jnp.dot(rep, pw[...], preferred_element_type=jnp.float32) + pb[...]


def _head_call(gid, sx, s1, s2, s3, s4, dke, pw, pb):
    return pl.pallas_call(
        _head_body,
        out_shape=jax.ShapeDtypeStruct((B, 10), jnp.float32),
    )(gid, sx, s1, s2, s3, s4, dke, pw, pb)


def kernel(x, params, edge_index, graph_ids):
    src = edge_index[1].astype(jnp.int32)
    dst = edge_index[0].astype(jnp.int32)

    npad = E_PAD - E
    pad_i = jnp.arange(npad, dtype=jnp.int32)
    src_p = jnp.concatenate([src, pad_i % 1024])
    dst_p = jnp.concatenate([dst, N + (pad_i % (ACC_ROWS - N))])
    g0 = E_PAD // (NC * NS * CHUNK)
    idx0 = jnp.concatenate([src_p.reshape(NC, NS, g0, 1, CHUNK),
                            dst_p.reshape(NC, NS, g0, 1, CHUNK)], axis=3)
    dst_q = dst_p.reshape(NS, G, 1, CHUNK)
    idx1 = jnp.stack([
        jnp.concatenate([(2 * src_p + c).reshape(NS, G, 1, CHUNK), dst_q], axis=2)
        for c in range(NC)])

    gid = graph_ids.reshape(1, N).astype(jnp.int32)
    h = x
    sums = [_gsum_call(gid, x)]
    for l in range(NL):
        if l == 0:
            pooled = _spmm_call(h, idx0, D_IN)
            combine = 'sum'
        else:
            pooled = _spmm_call(h.reshape(2 * N, HID // 2), idx1, HID // 2)
            combine = 'concat'
        h, s = _mlp_call(
            combine, gid, pooled[0], pooled[1], h,
            params[f'l{l}_W1'], params[f'l{l}_b1'].reshape(1, HID),
            params[f'l{l}_g1'].reshape(1, HID), params[f'l{l}_be1'].reshape(1, HID),
            params[f'l{l}_W2'], params[f'l{l}_b2'].reshape(1, HID),
            params[f'l{l}_go'].reshape(1, HID), params[f'l{l}_beo'].reshape(1, HID),
            params['eps'][l].reshape(1, 1),
        )
        sums.append(s)

    return _head_call(
        gid, sums[0], sums[1], sums[2], sums[3], sums[4],
        params['dke_W'], params['pred_W'], params['pred_b'].reshape(1, 10),
    )


# speedup vs baseline: 1.1171x; 1.0260x over previous
import functools

import jax
import jax.numpy as jnp
from jax import lax
from jax.experimental import pallas as pl
from jax.experimental.pallas import tpu as pltpu
from jax.experimental.pallas import tpu_sc as plsc

N = 10000
E = 320000
B = 64
D_IN = 128
HID = 256
NL = 4

NC = 2
NS = 16
CHUNK = 128
EPW = 20480
E_PAD = NS * EPW
G = EPW // CHUNK
NB = 2
ACC_ROWS = 10240


@functools.lru_cache(maxsize=None)
def _make_spmm(dh, g):
    mesh = plsc.VectorSubcoreMesh(core_axis_name="c", subcore_axis_name="s",
                                  num_cores=NC, num_subcores=NS)

    ib = 8
    ibg = 2 * ib
    ngroups = g // ibg

    @functools.partial(
        pl.kernel,
        out_type=jax.ShapeDtypeStruct((NC, ACC_ROWS, dh), jnp.float32),
        mesh=mesh,
        scratch_types=(
            [pltpu.VMEM((ib, 2, CHUNK), jnp.int32) for _ in range(2)]
            + [pltpu.VMEM((CHUNK, dh), jnp.float32) for _ in range(NB)]
            + [pltpu.VMEM_SHARED((ACC_ROWS, dh), jnp.float32)]
            + [pltpu.SemaphoreType.DMA for _ in range(2 * NB + 2)]
        ),
    )
    def spmm(h2, idx, out, *refs):
        idxbs = refs[0:2]
        rows = refs[2:2 + NB]
        acc = refs[2 + NB]
        gsem = refs[3 + NB:3 + 2 * NB]
        ssem = refs[3 + 2 * NB:3 + 3 * NB]
        isem = refs[2 + 3 * NB]
        isem1 = refs[3 + 3 * NB]
        idxb0, idxb1 = idxbs
        rows0 = rows[0]
        c = lax.axis_index("c")
        sid = lax.axis_index("s")

        def zrow(r, carry):
            for j in range(dh // 16):
                rows0[r, pl.ds(j * 16, 16)] = jnp.zeros((16,), jnp.float32)
            return carry

        lax.fori_loop(0, CHUNK, zrow, 0)
        zrows = ACC_ROWS // NS
        for z in range(zrows // CHUNK):
            pltpu.sync_copy(rows0, acc.at[pl.ds(sid * zrows + z * CHUNK, CHUNK)])
        plsc.subcore_barrier()

        def wait_rows(b):
            pltpu.make_async_copy(h2.at[pl.ds(0, CHUNK)], rows[b], gsem[b]).wait()

        def wait_scat(b):
            pltpu.make_async_copy(rows[b], acc.at[pl.ds(0, CHUNK)], ssem[b]).wait()

        pltpu.sync_copy(idx.at[c, sid, pl.ds(0, ib)], idxb0)
        pltpu.async_copy(idx.at[c, sid, pl.ds(ib, ib)], idxb1, isem1)
        pltpu.async_copy(h2.at[idxb0.at[0, 0]], rows[0], gsem[0])
        pltpu.async_copy(h2.at[idxb0.at[1, 0]], rows[1], gsem[1])

        def group(t, carry):
            for j2 in range(2):
                blk = idxbs[j2]
                if j2 == 1:
                    @pl.when(t < ngroups - 1)
                    def _pre0():
                        pltpu.async_copy(
                            idx.at[c, sid, pl.ds((t + 1) * ibg, ib)],
                            idxb0, isem)
                for u in range(ib):
                    p = j2 * ib + u
                    b = p % NB
                    if p == ib - 2:
                        pltpu.make_async_copy(
                            idx.at[c, sid, pl.ds(0, ib)], idxb1, isem1).wait()
                    wait_rows(b)
                    pltpu.sync_copy(rows[b], acc.at[blk.at[u, 1]], add=True)
                    if p < ibg - 2:
                        p2 = p + 2
                        blk2 = idxbs[p2 // ib]
                        pltpu.async_copy(h2.at[blk2.at[p2 % ib, 0]], rows[b],
                                         gsem[b])

            @pl.when(t < ngroups - 1)
            def _prefetch():
                base = (t + 1) * ibg
                pltpu.make_async_copy(idx.at[c, sid, pl.ds(base, ib)],
                                      idxb0, isem).wait()
                pltpu.async_copy(idx.at[c, sid, pl.ds(base + ib, ib)], idxb1,
                                 isem1)
                pltpu.async_copy(h2.at[idxb0.at[0, 0]], rows[0], gsem[0])
                pltpu.async_copy(h2.at[idxb0.at[1, 0]], rows[1], gsem[1])

            return carry

        lax.fori_loop(0, ngroups, group, 0)
        plsc.subcore_barrier()

        wrows = ACC_ROWS // NS
        pltpu.sync_copy(acc.at[pl.ds(sid * wrows, wrows)],
                        out.at[c, pl.ds(sid * wrows, wrows)])

    return spmm


def _spmm_call(h2, idx, dh):
    return _make_spmm(dh, idx.shape[2])(h2, idx)


def _mlp_body(gid, p0, p1, h, w1, b1, g1, be1, w2, b2, go, beo, eps, out,
              gout, *, combine):
    a = p0[pl.ds(0, N)]
    b = p1[pl.ds(0, N)]
    pooled = a + b if combine == 'sum' else jnp.concatenate([a, b], axis=-1)
    z = pooled + (1.0 + eps[0, 0]) * h[...]
    t = jnp.dot(z, w1[...], preferred_element_type=jnp.float32) + b1[...]
    m = jnp.mean(t, axis=0, keepdims=True)
    v = jnp.mean((t - m) * (t - m), axis=0, keepdims=True)
    t = (t - m) * lax.rsqrt(v + 1e-5) * g1[...] + be1[...]
    t = jnp.maximum(t, 0.0)
    u = jnp.dot(t, w2[...], preferred_element_type=jnp.float32) + b2[...]
    m2 = jnp.mean(u, axis=0, keepdims=True)
    v2 = jnp.mean((u - m2) * (u - m2), axis=0, keepdims=True)
    u = (u - m2) * lax.rsqrt(v2 + 1e-5) * go[...] + beo[...]
    hn = jnp.maximum(u, 0.0)
    out[...] = hn
    iot = lax.broadcasted_iota(jnp.int32, (B, N), 0)
    onehot = (gid[...] == iot).astype(jnp.float32)
    gout[...] = jnp.dot(onehot, hn, preferred_element_type=jnp.float32)


def _mlp_call(combine, gid, p0, p1, h, w1, b1, g1, be1, w2, b2, go, beo, eps):
    return pl.pallas_call(
        functools.partial(_mlp_body, combine=combine),
        out_shape=(jax.ShapeDtypeStruct((N, HID), jnp.float32),
                   jax.ShapeDtypeStruct((B, HID), jnp.float32)),
    )(gid, p0, p1, h, w1, b1, g1, be1, w2, b2, go, beo, eps)


def _gsum_body(gid, h, out):
    ids = gid[...]
    iot = lax.broadcasted_iota(jnp.int32, (B, N), 0)
    onehot = (ids == iot).astype(jnp.float32)
    out[...] = jnp.dot(onehot, h[...], preferred_element_type=jnp.float32)


def _gsum_call(gid, h):
    return pl.pallas_call(
        _gsum_body,
        out_shape=jax.ShapeDtypeStruct((B, h.shape[1]), jnp.float32),
    )(gid, h)


def _head_body(gid, sx, s1, s2, s3, s4, dke, pw, pb, out):
    ids = gid[...]
    iot = lax.broadcasted_iota(jnp.int32, (B, N), 0)
    onehot = (ids == iot).astype(jnp.float32)
    counts = jnp.sum(onehot, axis=1, keepdims=True)
    sums = jnp.concatenate(
        [sx[...], s1[...], s2[...], s3[...], s4[...]], axis=-1)
    means = sums / jnp.maximum(counts, 1.0)
    rep = jnp.dot(means, dke[...], preferred_element_type=jnp.float32)
    out[...] = jnp.dot(rep, pw[...], preferred_element_type=jnp.float32) + pb[...]


def _head_call(gid, sx, s1, s2, s3, s4, dke, pw, pb):
    return pl.pallas_call(
        _head_body,
        out_shape=jax.ShapeDtypeStruct((B, 10), jnp.float32),
    )(gid, sx, s1, s2, s3, s4, dke, pw, pb)


def kernel(x, params, edge_index, graph_ids):
    src = edge_index[1].astype(jnp.int32)
    dst = edge_index[0].astype(jnp.int32)

    npad = E_PAD - E
    pad_i = jnp.arange(npad, dtype=jnp.int32)
    src_p = jnp.concatenate([src, pad_i % 1024])
    dst_p = jnp.concatenate([dst, N + (pad_i % (ACC_ROWS - N))])
    g0 = E_PAD // (NC * NS * CHUNK)
    idx0 = jnp.concatenate([src_p.reshape(NC, NS, g0, 1, CHUNK),
                            dst_p.reshape(NC, NS, g0, 1, CHUNK)], axis=3)
    dst_q = dst_p.reshape(NS, G, 1, CHUNK)
    idx1 = jnp.stack([
        jnp.concatenate([(2 * src_p + c).reshape(NS, G, 1, CHUNK), dst_q], axis=2)
        for c in range(NC)])

    gid = graph_ids.reshape(1, N).astype(jnp.int32)
    h = x
    sums = [_gsum_call(gid, x)]
    for l in range(NL):
        if l == 0:
            pooled = _spmm_call(h, idx0, D_IN)
            combine = 'sum'
        else:
            pooled = _spmm_call(h.reshape(2 * N, HID // 2), idx1, HID // 2)
            combine = 'concat'
        h, s = _mlp_call(
            combine, gid, pooled[0], pooled[1], h,
            params[f'l{l}_W1'], params[f'l{l}_b1'].reshape(1, HID),
            params[f'l{l}_g1'].reshape(1, HID), params[f'l{l}_be1'].reshape(1, HID),
            params[f'l{l}_W2'], params[f'l{l}_b2'].reshape(1, HID),
            params[f'l{l}_go'].reshape(1, HID), params[f'l{l}_beo'].reshape(1, HID),
            params['eps'][l].reshape(1, 1),
        )
        sums.append(s)

    return _head_call(
        gid, sums[0], sums[1], sums[2], sums[3], sums[4],
        params['dke_W'], params['pred_W'], params['pred_b'].reshape(1, 10),
    )


# speedup vs baseline: 1.1722x; 1.0493x over previous
import functools

import jax
import jax.numpy as jnp
from jax import lax
from jax.experimental import pallas as pl
from jax.experimental.pallas import tpu as pltpu
from jax.experimental.pallas import tpu_sc as plsc

N = 10000
E = 320000
B = 64
D_IN = 128
HID = 256
NL = 4

NC = 2
NS = 16
CHUNK = 128
EPW = 20480
E_PAD = NS * EPW
G = EPW // CHUNK
NB = 2
ACC_ROWS = 10240


@functools.lru_cache(maxsize=None)
def _make_spmm(dh, g):
    mesh = plsc.VectorSubcoreMesh(core_axis_name="c", subcore_axis_name="s",
                                  num_cores=NC, num_subcores=NS)

    ib = 8
    ibg = 2 * ib
    ngroups = g // ibg

    @functools.partial(
        pl.kernel,
        out_type=jax.ShapeDtypeStruct((NC, ACC_ROWS, dh), jnp.float32),
        mesh=mesh,
        scratch_types=(
            [pltpu.VMEM((ib, 2, CHUNK), jnp.int32) for _ in range(2)]
            + [pltpu.VMEM((CHUNK, dh), jnp.float32) for _ in range(NB)]
            + [pltpu.VMEM_SHARED((ACC_ROWS, dh), jnp.float32)]
            + [pltpu.SemaphoreType.DMA for _ in range(2 * NB + 2)]
        ),
    )
    def spmm(h2, idx, out, *refs):
        idxbs = refs[0:2]
        rows = refs[2:2 + NB]
        acc = refs[2 + NB]
        gsem = refs[3 + NB:3 + 2 * NB]
        ssem = refs[3 + 2 * NB:3 + 3 * NB]
        isem = refs[2 + 3 * NB]
        isem1 = refs[3 + 3 * NB]
        idxb0, idxb1 = idxbs
        rows0 = rows[0]
        c = lax.axis_index("c")
        sid = lax.axis_index("s")

        def zrow(r, carry):
            for j in range(dh // 16):
                rows0[r, pl.ds(j * 16, 16)] = jnp.zeros((16,), jnp.float32)
            return carry

        lax.fori_loop(0, CHUNK, zrow, 0)
        zrows = ACC_ROWS // NS
        for z in range(zrows // CHUNK):
            pltpu.sync_copy(rows0, acc.at[pl.ds(sid * zrows + z * CHUNK, CHUNK)])
        plsc.subcore_barrier()

        def wait_rows(b):
            pltpu.make_async_copy(h2.at[pl.ds(0, CHUNK)], rows[b], gsem[b]).wait()

        def wait_scat(b):
            pltpu.make_async_copy(rows[b], acc.at[pl.ds(0, CHUNK)], ssem[b]).wait()

        pltpu.sync_copy(idx.at[c, sid, pl.ds(0, ib)], idxb0)
        pltpu.async_copy(idx.at[c, sid, pl.ds(ib, ib)], idxb1, isem1)
        pltpu.async_copy(h2.at[idxb0.at[0, 0]], rows[0], gsem[0])
        pltpu.async_copy(h2.at[idxb0.at[1, 0]], rows[1], gsem[1])

        def group(t, carry):
            not_last = t < ngroups - 1
            for j2 in range(2):
                blk = idxbs[j2]
                if j2 == 1:
                    @pl.when(not_last)
                    def _pre0():
                        pltpu.async_copy(
                            idx.at[c, sid, pl.ds((t + 1) * ibg, ib)],
                            idxb0, isem)
                for u in range(ib):
                    p = j2 * ib + u
                    b = p % NB
                    if p == ib - 2:
                        pltpu.make_async_copy(
                            idx.at[c, sid, pl.ds(0, ib)], idxb1, isem1).wait()
                    wait_rows(b)
                    pltpu.sync_copy(rows[b], acc.at[blk.at[u, 1]], add=True)
                    if p < ibg - 2:
                        p2 = p + 2
                        blk2 = idxbs[p2 // ib]
                        pltpu.async_copy(h2.at[blk2.at[p2 % ib, 0]], rows[b],
                                         gsem[b])
                    if p == ibg - 3:
                        @pl.when(not_last)
                        def _w0():
                            pltpu.make_async_copy(
                                idx.at[c, sid, pl.ds(0, ib)], idxb0,
                                isem).wait()
                    if p >= ibg - 2:
                        @pl.when(not_last)
                        def _g0():
                            pltpu.async_copy(h2.at[idxb0.at[p - (ibg - 2), 0]],
                                             rows[b], gsem[b])
                    if p == ibg - 1:
                        @pl.when(not_last)
                        def _i1():
                            pltpu.async_copy(
                                idx.at[c, sid, pl.ds((t + 1) * ibg + ib, ib)],
                                idxb1, isem1)

            return carry

        lax.fori_loop(0, ngroups, group, 0)
        plsc.subcore_barrier()

        wrows = ACC_ROWS // NS
        pltpu.sync_copy(acc.at[pl.ds(sid * wrows, wrows)],
                        out.at[c, pl.ds(sid * wrows, wrows)])

    return spmm


def _spmm_call(h2, idx, dh):
    return _make_spmm(dh, idx.shape[2])(h2, idx)


def _mlp_body(gid, p0, p1, h, w1, b1, g1, be1, w2, b2, go, beo, eps, out,
              gout, *, combine):
    a = p0[pl.ds(0, N)]
    b = p1[pl.ds(0, N)]
    pooled = a + b if combine == 'sum' else jnp.concatenate([a, b], axis=-1)
    z = pooled + (1.0 + eps[0, 0]) * h[...]
    t = jnp.dot(z, w1[...], preferred_element_type=jnp.float32) + b1[...]
    m = jnp.mean(t, axis=0, keepdims=True)
    v = jnp.mean((t - m) * (t - m), axis=0, keepdims=True)
    t = (t - m) * lax.rsqrt(v + 1e-5) * g1[...] + be1[...]
    t = jnp.maximum(t, 0.0)
    u = jnp.dot(t, w2[...], preferred_element_type=jnp.float32) + b2[...]
    m2 = jnp.mean(u, axis=0, keepdims=True)
    v2 = jnp.mean((u - m2) * (u - m2), axis=0, keepdims=True)
    u = (u - m2) * lax.rsqrt(v2 + 1e-5) * go[...] + beo[...]
    hn = jnp.maximum(u, 0.0)
    out[...] = hn
    iot = lax.broadcasted_iota(jnp.int32, (B, N), 0)
    onehot = (gid[...] == iot).astype(jnp.float32)
    gout[...] = jnp.dot(onehot, hn, preferred_element_type=jnp.float32)


def _mlp_call(combine, gid, p0, p1, h, w1, b1, g1, be1, w2, b2, go, beo, eps):
    return pl.pallas_call(
        functools.partial(_mlp_body, combine=combine),
        out_shape=(jax.ShapeDtypeStruct((N, HID), jnp.float32),
                   jax.ShapeDtypeStruct((B, HID), jnp.float32)),
    )(gid, p0, p1, h, w1, b1, g1, be1, w2, b2, go, beo, eps)


def _gsum_body(gid, h, out):
    ids = gid[...]
    iot = lax.broadcasted_iota(jnp.int32, (B, N), 0)
    onehot = (ids == iot).astype(jnp.float32)
    out[...] = jnp.dot(onehot, h[...], preferred_element_type=jnp.float32)


def _gsum_call(gid, h):
    return pl.pallas_call(
        _gsum_body,
        out_shape=jax.ShapeDtypeStruct((B, h.shape[1]), jnp.float32),
    )(gid, h)


def _head_body(gid, sx, s1, s2, s3, s4, dke, pw, pb, out):
    ids = gid[...]
    iot = lax.broadcasted_iota(jnp.int32, (B, N), 0)
    onehot = (ids == iot).astype(jnp.float32)
    counts = jnp.sum(onehot, axis=1, keepdims=True)
    sums = jnp.concatenate(
        [sx[...], s1[...], s2[...], s3[...], s4[...]], axis=-1)
    means = sums / jnp.maximum(counts, 1.0)
    rep = jnp.dot(means, dke[...], preferred_element_type=jnp.float32)
    out[...] = jnp.dot(rep, pw[...], preferred_element_type=jnp.float32) + pb[...]


def _head_call(gid, sx, s1, s2, s3, s4, dke, pw, pb):
    return pl.pallas_call(
        _head_body,
        out_shape=jax.ShapeDtypeStruct((B, 10), jnp.float32),
    )(gid, sx, s1, s2, s3, s4, dke, pw, pb)


def kernel(x, params, edge_index, graph_ids):
    src = edge_index[1].astype(jnp.int32)
    dst = edge_index[0].astype(jnp.int32)

    npad = E_PAD - E
    pad_i = jnp.arange(npad, dtype=jnp.int32)
    src_p = jnp.concatenate([src, pad_i % 1024])
    dst_p = jnp.concatenate([dst, N + (pad_i % (ACC_ROWS - N))])
    g0 = E_PAD // (NC * NS * CHUNK)
    idx0 = jnp.concatenate([src_p.reshape(NC, NS, g0, 1, CHUNK),
                            dst_p.reshape(NC, NS, g0, 1, CHUNK)], axis=3)
    dst_q = dst_p.reshape(NS, G, 1, CHUNK)
    idx1 = jnp.stack([
        jnp.concatenate([(2 * src_p + c).reshape(NS, G, 1, CHUNK), dst_q], axis=2)
        for c in range(NC)])

    gid = graph_ids.reshape(1, N).astype(jnp.int32)
    h = x
    sums = [_gsum_call(gid, x)]
    for l in range(NL):
        if l == 0:
            pooled = _spmm_call(h, idx0, D_IN)
            combine = 'sum'
        else:
            pooled = _spmm_call(h.reshape(2 * N, HID // 2), idx1, HID // 2)
            combine = 'concat'
        h, s = _mlp_call(
            combine, gid, pooled[0], pooled[1], h,
            params[f'l{l}_W1'], params[f'l{l}_b1'].reshape(1, HID),
            params[f'l{l}_g1'].reshape(1, HID), params[f'l{l}_be1'].reshape(1, HID),
            params[f'l{l}_W2'], params[f'l{l}_b2'].reshape(1, HID),
            params[f'l{l}_go'].reshape(1, HID), params[f'l{l}_beo'].reshape(1, HID),
            params['eps'][l].reshape(1, 1),
        )
        sums.append(s)

    return _head_call(
        gid, sums[0], sums[1], sums[2], sums[3], sums[4],
        params['dke_W'], params['pred_W'], params['pred_b'].reshape(1, 10),
    )


# speedup vs baseline: 1.1929x; 1.0177x over previous
import functools

import jax
import jax.numpy as jnp
from jax import lax
from jax.experimental import pallas as pl
from jax.experimental.pallas import tpu as pltpu
from jax.experimental.pallas import tpu_sc as plsc

N = 10000
E = 320000
B = 64
D_IN = 128
HID = 256
NL = 4

NC = 2
NS = 16
CHUNK = 128
EPW = 20480
E_PAD = NS * EPW
G = EPW // CHUNK
NB = 2
ACC_ROWS = 10240


@functools.lru_cache(maxsize=None)
def _make_spmm(dh, g, single_plane=False):
    mesh = plsc.VectorSubcoreMesh(core_axis_name="c", subcore_axis_name="s",
                                  num_cores=NC, num_subcores=NS)

    ib = 8
    ibg = 2 * ib
    ngroups = g // ibg

    out_shape = ((ACC_ROWS, NC * dh) if single_plane
                 else (NC, ACC_ROWS, dh))

    @functools.partial(
        pl.kernel,
        out_type=jax.ShapeDtypeStruct(out_shape, jnp.float32),
        mesh=mesh,
        scratch_types=(
            [pltpu.VMEM((ib, 2, CHUNK), jnp.int32) for _ in range(2)]
            + [pltpu.VMEM((CHUNK, dh), jnp.float32) for _ in range(NB)]
            + [pltpu.VMEM_SHARED((ACC_ROWS, dh), jnp.float32)]
            + [pltpu.SemaphoreType.DMA for _ in range(2 * NB + 2)]
        ),
    )
    def spmm(h2, idx, out, *refs):
        idxbs = refs[0:2]
        rows = refs[2:2 + NB]
        acc = refs[2 + NB]
        gsem = refs[3 + NB:3 + 2 * NB]
        ssem = refs[3 + 2 * NB:3 + 3 * NB]
        isem = refs[2 + 3 * NB]
        isem1 = refs[3 + 3 * NB]
        idxb0, idxb1 = idxbs
        rows0 = rows[0]
        c = lax.axis_index("c")
        sid = lax.axis_index("s")

        def zrow(r, carry):
            for j in range(dh // 16):
                rows0[r, pl.ds(j * 16, 16)] = jnp.zeros((16,), jnp.float32)
            return carry

        lax.fori_loop(0, CHUNK, zrow, 0)
        zrows = ACC_ROWS // NS
        for z in range(zrows // CHUNK):
            pltpu.sync_copy(rows0, acc.at[pl.ds(sid * zrows + z * CHUNK, CHUNK)])
        plsc.subcore_barrier()

        def wait_rows(b):
            pltpu.make_async_copy(h2.at[pl.ds(0, CHUNK)], rows[b], gsem[b]).wait()

        def wait_scat(b):
            pltpu.make_async_copy(rows[b], acc.at[pl.ds(0, CHUNK)], ssem[b]).wait()

        pltpu.sync_copy(idx.at[c, sid, pl.ds(0, ib)], idxb0)
        pltpu.async_copy(idx.at[c, sid, pl.ds(ib, ib)], idxb1, isem1)
        pltpu.async_copy(h2.at[idxb0.at[0, 0]], rows[0], gsem[0])
        pltpu.async_copy(h2.at[idxb0.at[1, 0]], rows[1], gsem[1])

        def group(t, carry):
            not_last = t < ngroups - 1
            for j2 in range(2):
                blk = idxbs[j2]
                if j2 == 1:
                    @pl.when(not_last)
                    def _pre0():
                        pltpu.async_copy(
                            idx.at[c, sid, pl.ds((t + 1) * ibg, ib)],
                            idxb0, isem)
                for u in range(ib):
                    p = j2 * ib + u
                    b = p % NB
                    if p == ib - 2:
                        pltpu.make_async_copy(
                            idx.at[c, sid, pl.ds(0, ib)], idxb1, isem1).wait()
                    wait_rows(b)
                    pltpu.sync_copy(rows[b], acc.at[blk.at[u, 1]], add=True)
                    if p < ibg - 2:
                        p2 = p + 2
                        blk2 = idxbs[p2 // ib]
                        pltpu.async_copy(h2.at[blk2.at[p2 % ib, 0]], rows[b],
                                         gsem[b])
                    if p == ibg - 3:
                        @pl.when(not_last)
                        def _w0():
                            pltpu.make_async_copy(
                                idx.at[c, sid, pl.ds(0, ib)], idxb0,
                                isem).wait()
                    if p >= ibg - 2:
                        @pl.when(not_last)
                        def _g0():
                            pltpu.async_copy(h2.at[idxb0.at[p - (ibg - 2), 0]],
                                             rows[b], gsem[b])
                    if p == ibg - 1:
                        @pl.when(not_last)
                        def _i1():
                            pltpu.async_copy(
                                idx.at[c, sid, pl.ds((t + 1) * ibg + ib, ib)],
                                idxb1, isem1)

            return carry

        lax.fori_loop(0, ngroups, group, 0)
        plsc.subcore_barrier()

        wrows = ACC_ROWS // NS
        if single_plane:
            col = pl.multiple_of(c * dh, 128)
            pltpu.sync_copy(acc.at[pl.ds(sid * wrows, wrows)],
                            out.at[pl.ds(sid * wrows, wrows), pl.ds(col, dh)])
        else:
            pltpu.sync_copy(acc.at[pl.ds(sid * wrows, wrows)],
                            out.at[c, pl.ds(sid * wrows, wrows)])

    return spmm


def _spmm_call(h2, idx, dh, single_plane=False):
    return _make_spmm(dh, idx.shape[2], single_plane)(h2, idx)


def _mlp_body(gid, p0, p1, h, w1, b1, g1, be1, w2, b2, go, beo, eps, out,
              gout, *, combine):
    if combine == 'sum':
        pooled = p0[pl.ds(0, N)] + p1[pl.ds(0, N)]
    else:
        pooled = p0[pl.ds(0, N)]
    z = pooled + (1.0 + eps[0, 0]) * h[...]
    t = jnp.dot(z, w1[...], preferred_element_type=jnp.float32) + b1[...]
    m = jnp.mean(t, axis=0, keepdims=True)
    v = jnp.mean((t - m) * (t - m), axis=0, keepdims=True)
    t = (t - m) * lax.rsqrt(v + 1e-5) * g1[...] + be1[...]
    t = jnp.maximum(t, 0.0)
    u = jnp.dot(t, w2[...], preferred_element_type=jnp.float32) + b2[...]
    m2 = jnp.mean(u, axis=0, keepdims=True)
    v2 = jnp.mean((u - m2) * (u - m2), axis=0, keepdims=True)
    u = (u - m2) * lax.rsqrt(v2 + 1e-5) * go[...] + beo[...]
    hn = jnp.maximum(u, 0.0)
    out[...] = hn
    iot = lax.broadcasted_iota(jnp.int32, (B, N), 0)
    onehot = (gid[...] == iot).astype(jnp.float32)
    gout[...] = jnp.dot(onehot, hn, preferred_element_type=jnp.float32)


def _mlp_call(combine, gid, p0, p1, h, w1, b1, g1, be1, w2, b2, go, beo, eps):
    return pl.pallas_call(
        functools.partial(_mlp_body, combine=combine),
        out_shape=(jax.ShapeDtypeStruct((N, HID), jnp.float32),
                   jax.ShapeDtypeStruct((B, HID), jnp.float32)),
    )(gid, p0, p1, h, w1, b1, g1, be1, w2, b2, go, beo, eps)


def _gsum_body(gid, h, out):
    ids = gid[...]
    iot = lax.broadcasted_iota(jnp.int32, (B, N), 0)
    onehot = (ids == iot).astype(jnp.float32)
    out[...] = jnp.dot(onehot, h[...], preferred_element_type=jnp.float32)


def _gsum_call(gid, h):
    return pl.pallas_call(
        _gsum_body,
        out_shape=jax.ShapeDtypeStruct((B, h.shape[1]), jnp.float32),
    )(gid, h)


def _head_body(gid, sx, s1, s2, s3, s4, dke, pw, pb, out):
    ids = gid[...]
    iot = lax.broadcasted_iota(jnp.int32, (B, N), 0)
    onehot = (ids == iot).astype(jnp.float32)
    counts = jnp.sum(onehot, axis=1, keepdims=True)
    sums = jnp.concatenate(
        [sx[...], s1[...], s2[...], s3[...], s4[...]], axis=-1)
    means = sums / jnp.maximum(counts, 1.0)
    rep = jnp.dot(means, dke[...], preferred_element_type=jnp.float32)
    out[...] = jnp.dot(rep, pw[...], preferred_element_type=jnp.float32) + pb[...]


def _head_call(gid, sx, s1, s2, s3, s4, dke, pw, pb):
    return pl.pallas_call(
        _head_body,
        out_shape=jax.ShapeDtypeStruct((B, 10), jnp.float32),
    )(gid, sx, s1, s2, s3, s4, dke, pw, pb)


def kernel(x, params, edge_index, graph_ids):
    src = edge_index[1].astype(jnp.int32)
    dst = edge_index[0].astype(jnp.int32)

    npad = E_PAD - E
    pad_i = jnp.arange(npad, dtype=jnp.int32)
    src_p = jnp.concatenate([src, pad_i % 1024])
    dst_p = jnp.concatenate([dst, N + (pad_i % (ACC_ROWS - N))])
    g0 = E_PAD // (NC * NS * CHUNK)
    idx0 = jnp.concatenate([src_p.reshape(NC, NS, g0, 1, CHUNK),
                            dst_p.reshape(NC, NS, g0, 1, CHUNK)], axis=3)
    dst_q = dst_p.reshape(NS, G, 1, CHUNK)
    idx1 = jnp.stack([
        jnp.concatenate([(2 * src_p + c).reshape(NS, G, 1, CHUNK), dst_q], axis=2)
        for c in range(NC)])

    gid = graph_ids.reshape(1, N).astype(jnp.int32)
    h = x
    sums = [_gsum_call(gid, x)]
    for l in range(NL):
        if l == 0:
            pooled = _spmm_call(h, idx0, D_IN)
            combine = 'sum'
            pp = (pooled[0], pooled[1])
        else:
            pooled = _spmm_call(h.reshape(2 * N, HID // 2), idx1, HID // 2,
                                single_plane=True)
            combine = 'single'
            pp = (pooled, pooled[:1])
        h, s = _mlp_call(
            combine, gid, pp[0], pp[1], h,
            params[f'l{l}_W1'], params[f'l{l}_b1'].reshape(1, HID),
            params[f'l{l}_g1'].reshape(1, HID), params[f'l{l}_be1'].reshape(1, HID),
            params[f'l{l}_W2'], params[f'l{l}_b2'].reshape(1, HID),
            params[f'l{l}_go'].reshape(1, HID), params[f'l{l}_beo'].reshape(1, HID),
            params['eps'][l].reshape(1, 1),
        )
        sums.append(s)

    return _head_call(
        gid, sums[0], sums[1], sums[2], sums[3], sums[4],
        params['dke_W'], params['pred_W'], params['pred_b'].reshape(1, 10),
    )
